# baseline (speedup 1.0000x reference)
import functools

import jax
import jax.numpy as jnp
from jax import lax
from jax.experimental import pallas as pl
from jax.experimental.pallas import tpu as pltpu
from jax.experimental.pallas import tpu_sc as plsc

N = 10000
E = 320000
D = 128
NP = 10240
NC = 2
NS = 16
NW = NC * NS
ROWS_PER_TILE = NP // NS
CH = 128
CPT = 80
ECHP = CPT * NW
G = 8
NGRP = CPT // G
NPAIR = CPT // 2

@functools.cache
def _mesh():
    return plsc.VectorSubcoreMesh(
        core_axis_name="c", subcore_axis_name="s",
        num_cores=NC, num_subcores=NS)


_SC_PARAMS = pltpu.CompilerParams(needs_layout_passes=False)


_DEG_CH = 2000


@functools.cache
def _inv_deg_kernel():
    return pl.kernel(
        _inv_deg_body,
        out_type=jax.ShapeDtypeStruct((NP,), jnp.float32),
        mesh=_mesh(),
        scratch_types=[
            pltpu.VMEM((_DEG_CH,), jnp.int32),
            pltpu.VMEM((NP,), jnp.float32),
            pltpu.VMEM((NS, ROWS_PER_TILE), jnp.float32),
            pltpu.VMEM((ROWS_PER_TILE,), jnp.float32),
            pltpu.VMEM_SHARED((NS, NP), jnp.float32),
        ],
        compiler_params=_SC_PARAMS,
    )


def _inv_deg_body(dst_hbm, inv_hbm, dbuf, deg_local, colbuf, inv_v, sdeg):
    cid = lax.axis_index("c")
    sid = lax.axis_index("s")
    zeros16 = jnp.zeros((16,), jnp.float32)
    ones16 = jnp.ones((16,), jnp.float32)

    def _zero(k, _):
        deg_local[pl.ds(k * 16, 16)] = zeros16
        return 0

    lax.fori_loop(0, NP // 16, _zero, 0)

    ebase = sid * (E // NS)

    def _chunk(c, _):
        pltpu.sync_copy(dst_hbm.at[pl.ds(ebase + c * _DEG_CH, _DEG_CH)], dbuf)

        def _hist(j, _):
            idx = dbuf[pl.ds(j * 16, 16)]
            plsc.addupdate_scatter(deg_local, [idx], ones16)
            return 0

        lax.fori_loop(0, _DEG_CH // 16, _hist, 0)
        return 0

    lax.fori_loop(0, (E // NS) // _DEG_CH, _chunk, 0)

    pltpu.sync_copy(deg_local, sdeg.at[sid])
    plsc.subcore_barrier()

    rb = sid * ROWS_PER_TILE
    for t in range(NS):
        pltpu.sync_copy(sdeg.at[t, pl.ds(rb, ROWS_PER_TILE)], colbuf.at[t])

    def _combine(j, _):
        acc = colbuf[0, pl.ds(j * 16, 16)]
        for t in range(1, NS):
            acc = acc + colbuf[t, pl.ds(j * 16, 16)]
        inv_v[pl.ds(j * 16, 16)] = 1.0 / jnp.maximum(acc, 1.0)
        return 0

    lax.fori_loop(0, ROWS_PER_TILE // 16, _combine, 0)

    half = ROWS_PER_TILE // 2
    off = cid * half
    pltpu.sync_copy(inv_v.at[pl.ds(off, half)], inv_hbm.at[pl.ds(rb + off, half)])


@functools.cache
def _agg_kernel():
    return pl.kernel(
        _agg_body,
        out_type=jax.ShapeDtypeStruct((NC, NP, D), jnp.float32),
        mesh=_mesh(),
        scratch_types=[
            pltpu.VMEM((3, G, CH), jnp.int32),
            pltpu.VMEM((3, G, CH), jnp.int32),
            pltpu.VMEM((2, CH, D), jnp.float32),
            pltpu.VMEM((ROWS_PER_TILE,), jnp.float32),
            pltpu.VMEM_SHARED((NP, D), jnp.float32),
            pltpu.SemaphoreType.DMA,
            pltpu.SemaphoreType.DMA,
            pltpu.SemaphoreType.DMA,
            pltpu.SemaphoreType.DMA,
            pltpu.SemaphoreType.DMA,
        ],
        compiler_params=_SC_PARAMS,
    )


def _agg_body(x_hbm, src_hbm, dst_hbm, inv_hbm, p_hbm,
              sidx, didx, rows, inv_v, shared, g0, g1, s0, s1, tsem):
    cid = lax.axis_index("c")
    sid = lax.axis_index("s")
    wid = cid * NS + sid
    cbase = CPT * wid
    rb = sid * ROWS_PER_TILE
    zeros16 = jnp.zeros((16,), jnp.float32)
    gsem = (g0, g1)
    ssem = (s0, s1)

    def _fire_stage(g):
        slot = g % 3
        pltpu.async_copy(src_hbm.at[pl.ds(cbase + g * G, G)],
                         sidx.at[slot], tsem)
        pltpu.async_copy(dst_hbm.at[pl.ds(cbase + g * G, G)],
                         didx.at[slot], tsem)

    def _wait_stage(g):
        slot = g % 3
        pltpu.make_async_copy(src_hbm.at[pl.ds(cbase, G)],
                              sidx.at[slot], tsem).wait()
        pltpu.make_async_copy(dst_hbm.at[pl.ds(cbase, G)],
                              didx.at[slot], tsem).wait()

    _fire_stage(0)
    _fire_stage(1)

    def _zero(r, _):
        for u in range(D // 16):
            rows[0, r, pl.ds(u * 16, 16)] = zeros16
        return 0

    lax.fori_loop(0, CH, _zero, 0)
    for z in range(ROWS_PER_TILE // CH):
        pltpu.sync_copy(rows.at[0], shared.at[pl.ds(rb + z * CH, CH)])
    pltpu.sync_copy(inv_hbm.at[pl.ds(rb, ROWS_PER_TILE)], inv_v)
    _wait_stage(0)
    plsc.subcore_barrier()

    def _fire_gather(c, b):
        pltpu.async_copy(x_hbm.at[sidx.at[(c // G) % 3, c % G]],
                         rows.at[b], gsem[b])

    def _wait_gather(b):
        pltpu.make_async_copy(x_hbm.at[sidx.at[0, 0]], rows.at[b],
                              gsem[b]).wait()

    def _fire_scatter(c, b):
        pltpu.async_copy(rows.at[b], shared.at[didx.at[(c // G) % 3, c % G]],
                         ssem[b], add=True)

    def _wait_scatter(b):
        pltpu.make_async_copy(rows.at[b], shared.at[didx.at[0, 0]],
                              ssem[b]).wait()

    _fire_gather(0, 0)
    for c in range(CPT):
        b = c % 2
        nb = 1 - b
        if c >= 1:
            _wait_scatter(nb)
        if c % G == 0 and c > 0 and c // G + 1 < NGRP:
            _fire_stage(c // G + 1)
        if c + 1 < CPT:
            if (c + 1) % G == 0:
                _wait_stage((c + 1) // G)
            _fire_gather(c + 1, nb)
        _wait_gather(b)
        _fire_scatter(c, b)
    _wait_scatter((CPT - 1) % 2)

    plsc.subcore_barrier()

    nz = ROWS_PER_TILE // CH
    for z in range(nz):
        b = z % 2
        if z >= 2:
            pltpu.make_async_copy(
                rows.at[b],
                p_hbm.at[cid, pl.ds(rb + (z - 2) * CH, CH)], gsem[b]).wait()
        pltpu.sync_copy(shared.at[pl.ds(rb + z * CH, CH)], rows.at[b])

        def _scale(g, _):
            sv = inv_v[pl.ds(z * CH + g * 16, 16)]
            for l in range(16):
                s = sv[l]
                r = g * 16 + l
                for u in range(D // 16):
                    rows[b, r, pl.ds(u * 16, 16)] = (
                        rows[b, r, pl.ds(u * 16, 16)] * s)
            return 0

        lax.fori_loop(0, CH // 16, _scale, 0)
        pltpu.async_copy(rows.at[b], p_hbm.at[cid, pl.ds(rb + z * CH, CH)],
                         gsem[b])
    for z in (nz - 2, nz - 1):
        pltpu.make_async_copy(
            rows.at[z % 2],
            p_hbm.at[cid, pl.ds(rb + z * CH, CH)], gsem[z % 2]).wait()


RB = 1024
NGRID = NP // RB


def _layer_body(act, x_ref, p_ref, ws_ref, wn_ref, b_ref, o_ref):
    hn = p_ref[0] + p_ref[1]
    h = (jnp.dot(x_ref[...], ws_ref[...], preferred_element_type=jnp.float32)
         + jnp.dot(hn, wn_ref[...], preferred_element_type=jnp.float32)
         + b_ref[...])
    if act:
        h = jnp.maximum(h, 0.0)
    o_ref[...] = h


_row_spec = pl.BlockSpec((RB, D), lambda i: (i, 0))
_p_spec = pl.BlockSpec((NC, RB, D), lambda i: (0, i, 0))
_full_spec = pl.BlockSpec((D, D), lambda i: (0, 0))
_b_spec = pl.BlockSpec((1, D), lambda i: (0, 0))


def _tc_layer(x, p, ws, wn, b2, act):
    return pl.pallas_call(
        functools.partial(_layer_body, act),
        grid=(NGRID,),
        in_specs=[_row_spec, _p_spec, _full_spec, _full_spec, _b_spec],
        out_specs=_row_spec,
        out_shape=jax.ShapeDtypeStruct((NP, D), jnp.float32),
    )(x, p, ws, wn, b2)


def _final_body(x_ref, p_ref, ws_ref, wn_ref, b_ref, o_ref, hg_ref):
    i = pl.program_id(0)
    hn = p_ref[0] + p_ref[1]
    h = (jnp.dot(x_ref[...], ws_ref[...], preferred_element_type=jnp.float32)
         + jnp.dot(hn, wn_ref[...], preferred_element_type=jnp.float32)
         + b_ref[...])
    o_ref[...] = h
    rid = lax.broadcasted_iota(jnp.int32, (RB, 1), 0) + i * RB
    hm = jnp.where(rid < N, h, 0.0)
    part = jnp.sum(hm, axis=0, keepdims=True)

    @pl.when(i == 0)
    def _():
        hg_ref[...] = jnp.zeros_like(hg_ref)

    hg_ref[...] += part

    @pl.when(i == NGRID - 1)
    def _():
        hg_ref[...] = hg_ref[...] * (1.0 / N)


def _tc_final(x, p, ws, wn, b2):
    return pl.pallas_call(
        _final_body,
        grid=(NGRID,),
        in_specs=[_row_spec, _p_spec, _full_spec, _full_spec, _b_spec],
        out_specs=[pl.BlockSpec((RB, D), lambda i: (i, 0)), _b_spec],
        out_shape=[jax.ShapeDtypeStruct((N, D), jnp.float32),
                   jax.ShapeDtypeStruct((1, D), jnp.float32)],
    )(x, p, ws, wn, b2)


def kernel(feature, edge_index, W_self_0, W_neigh_0, b_0,
           W_self_1, W_neigh_1, b_1, W_self_2, W_neigh_2, b_2):
    npad = ECHP * CH - E
    src = jnp.concatenate(
        [edge_index[0].astype(jnp.int32),
         jnp.zeros((npad,), jnp.int32)]).reshape(ECHP, CH)
    dst_flat = edge_index[1].astype(jnp.int32)
    dst = jnp.concatenate(
        [dst_flat, jnp.full((npad,), NP - 1, jnp.int32)]).reshape(ECHP, CH)
    xp = jnp.concatenate(
        [feature, jnp.zeros((NP - N, D), jnp.float32)], axis=0)

    inv = _inv_deg_kernel()(dst_flat)

    p = _agg_kernel()(xp, src, dst, inv)
    h = _tc_layer(xp, p, W_self_0, W_neigh_0, b_0.reshape(1, D), act=True)

    p = _agg_kernel()(h, src, dst, inv)
    h = _tc_layer(h, p, W_self_1, W_neigh_1, b_1.reshape(1, D), act=True)

    p = _agg_kernel()(h, src, dst, inv)
    h, hg = _tc_final(h, p, W_self_2, W_neigh_2, b_2.reshape(1, D))

    return (h, hg)


# speedup vs baseline: 3.4441x; 3.4441x over previous
import functools

import jax
import jax.numpy as jnp
from jax import lax
from jax.experimental import pallas as pl
from jax.experimental.pallas import tpu as pltpu
from jax.experimental.pallas import tpu_sc as plsc

N = 10000
E = 320000
D = 128
NP = 10240
NC = 2
NS = 16
NW = NC * NS
ROWS_PER_TILE = NP // NS
CH = 128
CPT = 80
ECHP = CPT * NW
G = 8
NGRP = CPT // G
NPAIR = CPT // 2

@functools.cache
def _mesh():
    return plsc.VectorSubcoreMesh(
        core_axis_name="c", subcore_axis_name="s",
        num_cores=NC, num_subcores=NS)


_SC_PARAMS = pltpu.CompilerParams(needs_layout_passes=False)


_DEG_CH = 2000


@functools.cache
def _inv_deg_kernel():
    return pl.kernel(
        _inv_deg_body,
        out_type=jax.ShapeDtypeStruct((NP,), jnp.float32),
        mesh=_mesh(),
        scratch_types=[
            pltpu.VMEM((_DEG_CH,), jnp.int32),
            pltpu.VMEM((NP,), jnp.float32),
            pltpu.VMEM((NS, ROWS_PER_TILE), jnp.float32),
            pltpu.VMEM((ROWS_PER_TILE,), jnp.float32),
            pltpu.VMEM_SHARED((NS, NP), jnp.float32),
        ],
        compiler_params=_SC_PARAMS,
    )


def _inv_deg_body(dst_hbm, inv_hbm, dbuf, deg_local, colbuf, inv_v, sdeg):
    cid = lax.axis_index("c")
    sid = lax.axis_index("s")
    zeros16 = jnp.zeros((16,), jnp.float32)
    ones16 = jnp.ones((16,), jnp.float32)

    def _zero(k, _):
        deg_local[pl.ds(k * 16, 16)] = zeros16
        return 0

    lax.fori_loop(0, NP // 16, _zero, 0)

    ebase = sid * (E // NS)

    def _chunk(c, _):
        pltpu.sync_copy(dst_hbm.at[pl.ds(ebase + c * _DEG_CH, _DEG_CH)], dbuf)

        def _hist(j, _):
            idx = dbuf[pl.ds(j * 16, 16)]
            plsc.addupdate_scatter(deg_local, [idx], ones16)
            return 0

        lax.fori_loop(0, _DEG_CH // 16, _hist, 0)
        return 0

    lax.fori_loop(0, (E // NS) // _DEG_CH, _chunk, 0)

    pltpu.sync_copy(deg_local, sdeg.at[sid])
    plsc.subcore_barrier()

    rb = sid * ROWS_PER_TILE
    for t in range(NS):
        pltpu.sync_copy(sdeg.at[t, pl.ds(rb, ROWS_PER_TILE)], colbuf.at[t])

    def _combine(j, _):
        acc = colbuf[0, pl.ds(j * 16, 16)]
        for t in range(1, NS):
            acc = acc + colbuf[t, pl.ds(j * 16, 16)]
        inv_v[pl.ds(j * 16, 16)] = 1.0 / jnp.maximum(acc, 1.0)
        return 0

    lax.fori_loop(0, ROWS_PER_TILE // 16, _combine, 0)

    half = ROWS_PER_TILE // 2
    off = cid * half
    pltpu.sync_copy(inv_v.at[pl.ds(off, half)], inv_hbm.at[pl.ds(rb + off, half)])


@functools.cache
def _agg_kernel():
    return pl.kernel(
        _agg_body,
        out_type=jax.ShapeDtypeStruct((NC, NP, D), jnp.float32),
        mesh=_mesh(),
        scratch_types=[
            pltpu.VMEM((3, G, CH), jnp.int32),
            pltpu.VMEM((3, G, CH), jnp.int32),
            pltpu.VMEM((2, CH, D), jnp.float32),
            pltpu.VMEM((ROWS_PER_TILE,), jnp.float32),
            pltpu.VMEM_SHARED((NP, D), jnp.float32),
            pltpu.SemaphoreType.DMA,
            pltpu.SemaphoreType.DMA,
            pltpu.SemaphoreType.DMA,
            pltpu.SemaphoreType.DMA,
            pltpu.SemaphoreType.DMA,
        ],
        compiler_params=_SC_PARAMS,
    )


def _agg_body(x_hbm, src_hbm, dst_hbm, inv_hbm, p_hbm,
              sidx, didx, rows, inv_v, shared, g0, g1, s0, s1, tsem):
    cid = lax.axis_index("c")
    sid = lax.axis_index("s")
    wid = cid * NS + sid
    cbase = CPT * wid
    rb = sid * ROWS_PER_TILE
    zeros16 = jnp.zeros((16,), jnp.float32)
    gsem = (g0, g1)
    ssem = (s0, s1)

    def _fire_stage(g):
        slot = g % 3
        pltpu.async_copy(src_hbm.at[pl.ds(cbase + g * G, G)],
                         sidx.at[slot], tsem)
        pltpu.async_copy(dst_hbm.at[pl.ds(cbase + g * G, G)],
                         didx.at[slot], tsem)

    def _wait_stage(g):
        slot = g % 3
        pltpu.make_async_copy(src_hbm.at[pl.ds(cbase, G)],
                              sidx.at[slot], tsem).wait()
        pltpu.make_async_copy(dst_hbm.at[pl.ds(cbase, G)],
                              didx.at[slot], tsem).wait()

    _fire_stage(0)
    _fire_stage(1)

    def _zero(r, _):
        for u in range(D // 16):
            rows[0, r, pl.ds(u * 16, 16)] = zeros16
        return 0

    lax.fori_loop(0, CH, _zero, 0)
    for z in range(ROWS_PER_TILE // CH):
        pltpu.sync_copy(rows.at[0], shared.at[pl.ds(rb + z * CH, CH)])
    pltpu.sync_copy(inv_hbm.at[pl.ds(rb, ROWS_PER_TILE)], inv_v)
    _wait_stage(0)
    plsc.subcore_barrier()

    def _fire_gather(c, b):
        pltpu.async_copy(x_hbm.at[sidx.at[(c // G) % 3, c % G]],
                         rows.at[b], gsem[b])

    def _wait_gather(b):
        pltpu.make_async_copy(x_hbm.at[sidx.at[0, 0]], rows.at[b],
                              gsem[b]).wait()

    def _fire_scatter(c, b):
        pltpu.async_copy(rows.at[b], shared.at[didx.at[(c // G) % 3, c % G]],
                         ssem[b], add=True)

    def _wait_scatter(b):
        pltpu.make_async_copy(rows.at[b], shared.at[didx.at[0, 0]],
                              ssem[b]).wait()

    _fire_gather(0, 0)
    for c in range(CPT):
        b = c % 2
        nb = 1 - b
        if c >= 1:
            _wait_scatter(nb)
        if c % G == 0 and c > 0 and c // G + 1 < NGRP:
            _fire_stage(c // G + 1)
        if c + 1 < CPT:
            if (c + 1) % G == 0:
                _wait_stage((c + 1) // G)
            _fire_gather(c + 1, nb)
        _wait_gather(b)
        _fire_scatter(c, b)
    _wait_scatter((CPT - 1) % 2)

    plsc.subcore_barrier()

    nz = ROWS_PER_TILE // CH
    for z in range(nz):
        b = z % 2
        if z >= 2:
            pltpu.make_async_copy(
                rows.at[b],
                p_hbm.at[cid, pl.ds(rb + (z - 2) * CH, CH)], gsem[b]).wait()
        pltpu.sync_copy(shared.at[pl.ds(rb + z * CH, CH)], rows.at[b])

        def _scale(g, _):
            sv = inv_v[pl.ds(z * CH + g * 16, 16)]
            for l in range(16):
                s = sv[l]
                r = g * 16 + l
                for u in range(D // 16):
                    rows[b, r, pl.ds(u * 16, 16)] = (
                        rows[b, r, pl.ds(u * 16, 16)] * s)
            return 0

        lax.fori_loop(0, CH // 16, _scale, 0)
        pltpu.async_copy(rows.at[b], p_hbm.at[cid, pl.ds(rb + z * CH, CH)],
                         gsem[b])
    for z in (nz - 2, nz - 1):
        pltpu.make_async_copy(
            rows.at[z % 2],
            p_hbm.at[cid, pl.ds(rb + z * CH, CH)], gsem[z % 2]).wait()


RB = 1024
NGRID = NP // RB


def _layer_body(act, x_ref, p_ref, ws_ref, wn_ref, b_ref, o_ref):
    hn = p_ref[0] + p_ref[1]
    h = (jnp.dot(x_ref[...], ws_ref[...], preferred_element_type=jnp.float32)
         + jnp.dot(hn, wn_ref[...], preferred_element_type=jnp.float32)
         + b_ref[...])
    if act:
        h = jnp.maximum(h, 0.0)
    o_ref[...] = h


_row_spec = pl.BlockSpec((RB, D), lambda i: (i, 0))
_p_spec = pl.BlockSpec((NC, RB, D), lambda i: (0, i, 0))
_full_spec = pl.BlockSpec((D, D), lambda i: (0, 0))
_b_spec = pl.BlockSpec((1, D), lambda i: (0, 0))


def _tc_layer(x, p, ws, wn, b2, act):
    return pl.pallas_call(
        functools.partial(_layer_body, act),
        grid=(NGRID,),
        in_specs=[_row_spec, _p_spec, _full_spec, _full_spec, _b_spec],
        out_specs=_row_spec,
        out_shape=jax.ShapeDtypeStruct((NP, D), jnp.float32),
    )(x, p, ws, wn, b2)


def _final_body(x_ref, p_ref, ws_ref, wn_ref, b_ref, o_ref, hg_ref):
    i = pl.program_id(0)
    hn = p_ref[0] + p_ref[1]
    h = (jnp.dot(x_ref[...], ws_ref[...], preferred_element_type=jnp.float32)
         + jnp.dot(hn, wn_ref[...], preferred_element_type=jnp.float32)
         + b_ref[...])
    o_ref[...] = h
    rid = lax.broadcasted_iota(jnp.int32, (RB, 1), 0) + i * RB
    hm = jnp.where(rid < N, h, 0.0)
    part = jnp.sum(hm, axis=0, keepdims=True)

    @pl.when(i == 0)
    def _():
        hg_ref[...] = jnp.zeros_like(hg_ref)

    hg_ref[...] += part

    @pl.when(i == NGRID - 1)
    def _():
        hg_ref[...] = hg_ref[...] * (1.0 / N)


def _tc_final(x, p, ws, wn, b2):
    return pl.pallas_call(
        _final_body,
        grid=(NGRID,),
        in_specs=[_row_spec, _p_spec, _full_spec, _full_spec, _b_spec],
        out_specs=[pl.BlockSpec((RB, D), lambda i: (i, 0)), _b_spec],
        out_shape=[jax.ShapeDtypeStruct((N, D), jnp.float32),
                   jax.ShapeDtypeStruct((1, D), jnp.float32)],
    )(x, p, ws, wn, b2)


def kernel(feature, edge_index, W_self_0, W_neigh_0, b_0,
           W_self_1, W_neigh_1, b_1, W_self_2, W_neigh_2, b_2):
    npad = ECHP * CH - E
    pad_iota = jnp.arange(npad, dtype=jnp.int32)
    src = jnp.concatenate(
        [edge_index[0].astype(jnp.int32),
         pad_iota % N]).reshape(ECHP, CH)
    dst_flat = edge_index[1].astype(jnp.int32)
    dst = jnp.concatenate(
        [dst_flat, N + pad_iota % (NP - N)]).reshape(ECHP, CH)
    xp = jnp.concatenate(
        [feature, jnp.zeros((NP - N, D), jnp.float32)], axis=0)

    inv = _inv_deg_kernel()(dst_flat)

    p = _agg_kernel()(xp, src, dst, inv)
    h = _tc_layer(xp, p, W_self_0, W_neigh_0, b_0.reshape(1, D), act=True)

    p = _agg_kernel()(h, src, dst, inv)
    h = _tc_layer(h, p, W_self_1, W_neigh_1, b_1.reshape(1, D), act=True)

    p = _agg_kernel()(h, src, dst, inv)
    h, hg = _tc_final(h, p, W_self_2, W_neigh_2, b_2.reshape(1, D))

    return (h, hg)


# speedup vs baseline: 3.4712x; 1.0079x over previous
import functools

import jax
import jax.numpy as jnp
from jax import lax
from jax.experimental import pallas as pl
from jax.experimental.pallas import tpu as pltpu
from jax.experimental.pallas import tpu_sc as plsc

N = 10000
E = 320000
D = 128
NP = 10240
NC = 2
NS = 16
NW = NC * NS
ROWS_PER_TILE = NP // NS
CH = 128
CPT = 80
ECHP = CPT * NW
G = 8
NGRP = CPT // G
NPAIR = CPT // 2

@functools.cache
def _mesh():
    return plsc.VectorSubcoreMesh(
        core_axis_name="c", subcore_axis_name="s",
        num_cores=NC, num_subcores=NS)


_SC_PARAMS = pltpu.CompilerParams(needs_layout_passes=False)


_DEG_CH = 2000


@functools.cache
def _inv_deg_kernel():
    return pl.kernel(
        _inv_deg_body,
        out_type=jax.ShapeDtypeStruct((NP,), jnp.float32),
        mesh=_mesh(),
        scratch_types=[
            pltpu.VMEM((_DEG_CH,), jnp.int32),
            pltpu.VMEM((NP,), jnp.float32),
            pltpu.VMEM((NS, ROWS_PER_TILE), jnp.float32),
            pltpu.VMEM((ROWS_PER_TILE,), jnp.float32),
            pltpu.VMEM_SHARED((NS, NP), jnp.float32),
        ],
        compiler_params=_SC_PARAMS,
    )


def _inv_deg_body(dst_hbm, inv_hbm, dbuf, deg_local, colbuf, inv_v, sdeg):
    cid = lax.axis_index("c")
    sid = lax.axis_index("s")
    zeros16 = jnp.zeros((16,), jnp.float32)
    ones16 = jnp.ones((16,), jnp.float32)

    def _zero(k, _):
        deg_local[pl.ds(k * 16, 16)] = zeros16
        return 0

    lax.fori_loop(0, NP // 16, _zero, 0)

    ebase = sid * (E // NS)

    def _chunk(c, _):
        pltpu.sync_copy(dst_hbm.at[pl.ds(ebase + c * _DEG_CH, _DEG_CH)], dbuf)

        def _hist(j, _):
            idx = dbuf[pl.ds(j * 16, 16)]
            plsc.addupdate_scatter(deg_local, [idx], ones16)
            return 0

        lax.fori_loop(0, _DEG_CH // 16, _hist, 0)
        return 0

    lax.fori_loop(0, (E // NS) // _DEG_CH, _chunk, 0)

    pltpu.sync_copy(deg_local, sdeg.at[sid])
    plsc.subcore_barrier()

    rb = sid * ROWS_PER_TILE
    for t in range(NS):
        pltpu.sync_copy(sdeg.at[t, pl.ds(rb, ROWS_PER_TILE)], colbuf.at[t])

    def _combine(j, _):
        acc = colbuf[0, pl.ds(j * 16, 16)]
        for t in range(1, NS):
            acc = acc + colbuf[t, pl.ds(j * 16, 16)]
        inv_v[pl.ds(j * 16, 16)] = 1.0 / jnp.maximum(acc, 1.0)
        return 0

    lax.fori_loop(0, ROWS_PER_TILE // 16, _combine, 0)

    half = ROWS_PER_TILE // 2
    off = cid * half
    pltpu.sync_copy(inv_v.at[pl.ds(off, half)], inv_hbm.at[pl.ds(rb + off, half)])


@functools.cache
def _agg_kernel():
    return pl.kernel(
        _agg_body,
        out_type=jax.ShapeDtypeStruct((NC, NP, D), jnp.float32),
        mesh=_mesh(),
        scratch_types=[
            pltpu.VMEM((3, G, CH), jnp.int32),
            pltpu.VMEM((3, G, CH), jnp.int32),
            pltpu.VMEM((2, CH, D), jnp.float32),
            pltpu.VMEM((ROWS_PER_TILE,), jnp.float32),
            pltpu.VMEM_SHARED((NP, D), jnp.float32),
            pltpu.SemaphoreType.DMA,
            pltpu.SemaphoreType.DMA,
            pltpu.SemaphoreType.DMA,
            pltpu.SemaphoreType.DMA,
            pltpu.SemaphoreType.DMA,
        ],
        compiler_params=_SC_PARAMS,
    )


def _agg_body(x_hbm, src_hbm, dst_hbm, inv_hbm, p_hbm,
              sidx, didx, rows, inv_v, shared, g0, g1, s0, s1, tsem):
    cid = lax.axis_index("c")
    sid = lax.axis_index("s")
    wid = cid * NS + sid
    cbase = CPT * wid
    rb = sid * ROWS_PER_TILE
    zeros16 = jnp.zeros((16,), jnp.float32)
    gsem = (g0, g1)
    ssem = (s0, s1)

    def _fire_stage(g):
        slot = g % 3
        pltpu.async_copy(src_hbm.at[pl.ds(cbase + g * G, G)],
                         sidx.at[slot], tsem)
        pltpu.async_copy(dst_hbm.at[pl.ds(cbase + g * G, G)],
                         didx.at[slot], tsem)

    def _wait_stage(g):
        slot = g % 3
        pltpu.make_async_copy(src_hbm.at[pl.ds(cbase, G)],
                              sidx.at[slot], tsem).wait()
        pltpu.make_async_copy(dst_hbm.at[pl.ds(cbase, G)],
                              didx.at[slot], tsem).wait()

    _fire_stage(0)
    _fire_stage(1)

    def _zero(r, _):
        for u in range(D // 16):
            rows[0, r, pl.ds(u * 16, 16)] = zeros16
        return 0

    lax.fori_loop(0, CH, _zero, 0)
    for z in range(ROWS_PER_TILE // CH):
        pltpu.sync_copy(rows.at[0], shared.at[pl.ds(rb + z * CH, CH)])
    pltpu.sync_copy(inv_hbm.at[pl.ds(rb, ROWS_PER_TILE)], inv_v)
    _wait_stage(0)
    plsc.subcore_barrier()

    def _fire_gather(c, b):
        pltpu.async_copy(x_hbm.at[sidx.at[(c // G) % 3, c % G]],
                         rows.at[b], gsem[b])

    def _wait_gather(b):
        pltpu.make_async_copy(x_hbm.at[sidx.at[0, 0]], rows.at[b],
                              gsem[b]).wait()

    def _fire_scatter(c, b):
        pltpu.async_copy(rows.at[b], shared.at[didx.at[(c // G) % 3, c % G]],
                         ssem[b], add=True)

    def _wait_scatter(b):
        pltpu.make_async_copy(rows.at[b], shared.at[didx.at[0, 0]],
                              ssem[b]).wait()

    _fire_gather(0, 0)
    for c in range(CPT):
        b = c % 2
        nb = 1 - b
        if c >= 1:
            _wait_scatter(nb)
        if c % G == 0 and c > 0 and c // G + 1 < NGRP:
            _fire_stage(c // G + 1)
        if c + 1 < CPT:
            if (c + 1) % G == 0:
                _wait_stage((c + 1) // G)
            _fire_gather(c + 1, nb)
        _wait_gather(b)
        _fire_scatter(c, b)
    _wait_scatter((CPT - 1) % 2)

    plsc.subcore_barrier()

    nz = ROWS_PER_TILE // CH

    def _spmem_rd(z, b, wait):
        cp = (pltpu.make_async_copy if wait else pltpu.async_copy)
        d = cp(shared.at[pl.ds(rb + z * CH, CH)], rows.at[b], ssem[b])
        if wait:
            d.wait()

    def _hbm_wr(z, b, wait):
        cp = (pltpu.make_async_copy if wait else pltpu.async_copy)
        d = cp(rows.at[b], p_hbm.at[cid, pl.ds(rb + z * CH, CH)], gsem[b])
        if wait:
            d.wait()

    _spmem_rd(0, 0, False)
    for z in range(nz):
        b = z % 2
        if z + 1 < nz:
            if z >= 1:
                _hbm_wr(z - 1, 1 - b, True)
            _spmem_rd(z + 1, 1 - b, False)
        _spmem_rd(z, b, True)

        def _scale(g, _):
            sv = inv_v[pl.ds(z * CH + g * 16, 16)]
            for l in range(16):
                s = sv[l]
                r = g * 16 + l
                for u in range(D // 16):
                    rows[b, r, pl.ds(u * 16, 16)] = (
                        rows[b, r, pl.ds(u * 16, 16)] * s)
            return 0

        lax.fori_loop(0, CH // 16, _scale, 0)
        _hbm_wr(z, b, False)
    _hbm_wr(nz - 2, (nz - 2) % 2, True)
    _hbm_wr(nz - 1, (nz - 1) % 2, True)


RB = 1024
NGRID = NP // RB


def _layer_body(act, x_ref, p_ref, ws_ref, wn_ref, b_ref, o_ref):
    hn = p_ref[0] + p_ref[1]
    h = (jnp.dot(x_ref[...], ws_ref[...], preferred_element_type=jnp.float32)
         + jnp.dot(hn, wn_ref[...], preferred_element_type=jnp.float32)
         + b_ref[...])
    if act:
        h = jnp.maximum(h, 0.0)
    o_ref[...] = h


_row_spec = pl.BlockSpec((RB, D), lambda i: (i, 0))
_p_spec = pl.BlockSpec((NC, RB, D), lambda i: (0, i, 0))
_full_spec = pl.BlockSpec((D, D), lambda i: (0, 0))
_b_spec = pl.BlockSpec((1, D), lambda i: (0, 0))


def _tc_layer(x, p, ws, wn, b2, act):
    return pl.pallas_call(
        functools.partial(_layer_body, act),
        grid=(NGRID,),
        in_specs=[_row_spec, _p_spec, _full_spec, _full_spec, _b_spec],
        out_specs=_row_spec,
        out_shape=jax.ShapeDtypeStruct((NP, D), jnp.float32),
    )(x, p, ws, wn, b2)


def _final_body(x_ref, p_ref, ws_ref, wn_ref, b_ref, o_ref, hg_ref):
    i = pl.program_id(0)
    hn = p_ref[0] + p_ref[1]
    h = (jnp.dot(x_ref[...], ws_ref[...], preferred_element_type=jnp.float32)
         + jnp.dot(hn, wn_ref[...], preferred_element_type=jnp.float32)
         + b_ref[...])
    o_ref[...] = h
    rid = lax.broadcasted_iota(jnp.int32, (RB, 1), 0) + i * RB
    hm = jnp.where(rid < N, h, 0.0)
    part = jnp.sum(hm, axis=0, keepdims=True)

    @pl.when(i == 0)
    def _():
        hg_ref[...] = jnp.zeros_like(hg_ref)

    hg_ref[...] += part

    @pl.when(i == NGRID - 1)
    def _():
        hg_ref[...] = hg_ref[...] * (1.0 / N)


def _tc_final(x, p, ws, wn, b2):
    return pl.pallas_call(
        _final_body,
        grid=(NGRID,),
        in_specs=[_row_spec, _p_spec, _full_spec, _full_spec, _b_spec],
        out_specs=[pl.BlockSpec((RB, D), lambda i: (i, 0)), _b_spec],
        out_shape=[jax.ShapeDtypeStruct((N, D), jnp.float32),
                   jax.ShapeDtypeStruct((1, D), jnp.float32)],
    )(x, p, ws, wn, b2)


def kernel(feature, edge_index, W_self_0, W_neigh_0, b_0,
           W_self_1, W_neigh_1, b_1, W_self_2, W_neigh_2, b_2):
    npad = ECHP * CH - E
    pad_iota = jnp.arange(npad, dtype=jnp.int32)
    src = jnp.concatenate(
        [edge_index[0].astype(jnp.int32),
         pad_iota % N]).reshape(ECHP, CH)
    dst_flat = edge_index[1].astype(jnp.int32)
    dst = jnp.concatenate(
        [dst_flat, N + pad_iota % (NP - N)]).reshape(ECHP, CH)
    inv = _inv_deg_kernel()(dst_flat)

    p = _agg_kernel()(feature, src, dst, inv)
    h = _tc_layer(feature, p, W_self_0, W_neigh_0, b_0.reshape(1, D),
                  act=True)

    p = _agg_kernel()(h, src, dst, inv)
    h = _tc_layer(h, p, W_self_1, W_neigh_1, b_1.reshape(1, D), act=True)

    p = _agg_kernel()(h, src, dst, inv)
    h, hg = _tc_final(h, p, W_self_2, W_neigh_2, b_2.reshape(1, D))

    return (h, hg)
